# baseline (speedup 1.0000x reference)
import functools

import jax
import jax.numpy as jnp
from jax import lax
from jax.experimental import pallas as pl
from jax.experimental.pallas import tpu as pltpu
from jax.experimental.pallas import tpu_sc as plsc

N = 10000
E = 320000
D = 128

NC, NS, L = 2, 16, 16
NW = NC * NS
EPW = E // NW
C = 80
NCHUNK = EPW // C
NP = 10240
RPT = NP // NS

_mesh = plsc.VectorSubcoreMesh(core_axis_name="c", subcore_axis_name="s")
_params = pltpu.CompilerParams(needs_layout_passes=False)


@functools.partial(
    pl.kernel,
    out_type=jax.ShapeDtypeStruct((NW, 1, N), jnp.float32),
    mesh=_mesh,
    compiler_params=_params,
    scratch_types=[
        pltpu.VMEM((C,), jnp.int32),
        pltpu.VMEM((1, N), jnp.float32),
    ],
)
def _deg_kernel(dst_hbm, out_hbm, idx_d, acc):
    cid = lax.axis_index("c")
    sid = lax.axis_index("s")
    wid = sid * NC + cid
    zf = jnp.zeros((L,), jnp.float32)

    def zinit(i, _):
        acc[0, pl.ds(i * L, L)] = zf
        return 0

    lax.fori_loop(0, N // L, zinit, 0)

    row0 = jnp.zeros((L,), jnp.int32)
    onesv = jnp.ones((L,), jnp.float32)

    def body(g, _):
        base = wid * EPW + g * C
        pltpu.sync_copy(dst_hbm.at[pl.ds(base, C)], idx_d)
        for k in range(C // L):
            iv = idx_d[pl.ds(k * L, L)]
            plsc.addupdate_scatter(acc, [row0, iv], onesv)
        return 0

    lax.fori_loop(0, NCHUNK, body, 0)
    pltpu.sync_copy(acc, out_hbm.at[wid])


@functools.partial(
    pl.kernel,
    out_type=(jax.ShapeDtypeStruct((NC, NP, D), jnp.float32),
              jax.ShapeDtypeStruct((NW, 1, N), jnp.float32)),
    mesh=_mesh,
    compiler_params=_params,
    scratch_types=[
        pltpu.VMEM((C,), jnp.int32),
        pltpu.VMEM((C,), jnp.int32),
        pltpu.VMEM((C, D), jnp.float32),
        pltpu.VMEM((N,), jnp.float32),
        pltpu.VMEM((1, N), jnp.float32),
        pltpu.VMEM_SHARED((NP, D), jnp.float32),
        pltpu.SemaphoreType.DMA,
    ],
)
def _agg_w_kernel(z_hbm, src_hbm, dst_hbm, dinv_hbm, zeros_hbm,
                  out_hbm, wout_hbm, idx_s, idx_d, rows, dinv_v, wacc,
                  acc, sem):
    cid = lax.axis_index("c")
    sid = lax.axis_index("s")
    wid = sid * NC + cid
    sl = pl.ds(sid * RPT, RPT)
    @pl.when(cid == 0)
    def _():
        pltpu.sync_copy(z_hbm.at[sl], acc.at[sl])

    @pl.when(cid != 0)
    def _():
        pltpu.sync_copy(zeros_hbm.at[sl], acc.at[sl])

    pltpu.sync_copy(dinv_hbm, dinv_v)
    zf = jnp.zeros((L,), jnp.float32)

    def zinit(i, _):
        wacc[0, pl.ds(i * L, L)] = zf
        return 0

    lax.fori_loop(0, N // L, zinit, 0)
    plsc.subcore_barrier()

    row0 = jnp.zeros((L,), jnp.int32)

    def body(g, _):
        base = wid * EPW + g * C
        pltpu.sync_copy(src_hbm.at[pl.ds(base, C)], idx_s)
        pltpu.sync_copy(dst_hbm.at[pl.ds(base, C)], idx_d)
        pltpu.async_copy(z_hbm.at[idx_s], rows, sem).wait()
        pltpu.sync_copy(rows, acc.at[idx_d], add=True)
        for k in range(C // L):
            iv_d = idx_d[pl.ds(k * L, L)]
            iv_s = idx_s[pl.ds(k * L, L)]
            dv = plsc.load_gather(dinv_v, [iv_d])
            plsc.addupdate_scatter(wacc, [row0, iv_s], dv)
        return 0

    lax.fori_loop(0, NCHUNK, body, 0)
    plsc.subcore_barrier()
    pltpu.sync_copy(acc.at[sl], out_hbm.at[cid, sl])
    pltpu.sync_copy(wacc, wout_hbm.at[wid])


@functools.partial(
    pl.kernel,
    out_type=jax.ShapeDtypeStruct((NC, NP, D), jnp.float32),
    mesh=_mesh,
    compiler_params=_params,
    scratch_types=[
        pltpu.VMEM((C,), jnp.int32),
        pltpu.VMEM((C,), jnp.int32),
        pltpu.VMEM((C, D), jnp.float32),
        pltpu.VMEM_SHARED((NP, D), jnp.float32),
        pltpu.SemaphoreType.DMA,
    ],
)
def _agg_kernel(z_hbm, src_hbm, dst_hbm, zeros_hbm,
                out_hbm, idx_s, idx_d, rows, acc, sem):
    cid = lax.axis_index("c")
    sid = lax.axis_index("s")
    wid = sid * NC + cid
    sl = pl.ds(sid * RPT, RPT)
    @pl.when(cid == 0)
    def _():
        pltpu.sync_copy(z_hbm.at[sl], acc.at[sl])

    @pl.when(cid != 0)
    def _():
        pltpu.sync_copy(zeros_hbm.at[sl], acc.at[sl])

    plsc.subcore_barrier()

    def body(g, _):
        base = wid * EPW + g * C
        pltpu.sync_copy(src_hbm.at[pl.ds(base, C)], idx_s)
        pltpu.sync_copy(dst_hbm.at[pl.ds(base, C)], idx_d)
        pltpu.async_copy(z_hbm.at[idx_s], rows, sem).wait()
        pltpu.sync_copy(rows, acc.at[idx_d], add=True)
        return 0

    lax.fori_loop(0, NCHUNK, body, 0)
    plsc.subcore_barrier()
    pltpu.sync_copy(acc.at[sl], out_hbm.at[cid, sl])


BR = 640
GRID = NP // BR
_HI = lax.Precision.HIGHEST


def _mm1_body(x_ref, degp_ref, w_ref, z_ref, dinv_ref):
    d = lax.rsqrt(jnp.sum(degp_ref[...], axis=0) + 1.0)[:, None]
    z_ref[...] = jnp.dot(x_ref[...], w_ref[...], precision=_HI,
                         preferred_element_type=jnp.float32) * d
    dinv_ref[...] = d


_mm1 = pl.pallas_call(
    _mm1_body,
    grid=(GRID,),
    in_specs=[pl.BlockSpec((BR, D), lambda i: (i, 0)),
              pl.BlockSpec((NW, BR), lambda i: (0, i)),
              pl.BlockSpec((D, D), lambda i: (0, 0))],
    out_specs=[pl.BlockSpec((BR, D), lambda i: (i, 0)),
               pl.BlockSpec((BR, 1), lambda i: (i, 0))],
    out_shape=[jax.ShapeDtypeStruct((NP, D), jnp.float32),
               jax.ShapeDtypeStruct((NP, 1), jnp.float32)],
)


def _mm2_body(p_ref, dinv_ref, b1_ref, w_ref, z_ref):
    d = dinv_ref[...]
    h1 = jnp.maximum((p_ref[0] + p_ref[1]) * d + b1_ref[...], 0.0)
    z_ref[...] = jnp.dot(h1, w_ref[...], precision=_HI,
                         preferred_element_type=jnp.float32) * d


_mm2 = pl.pallas_call(
    _mm2_body,
    grid=(GRID,),
    in_specs=[pl.BlockSpec((NC, BR, D), lambda i: (0, i, 0)),
              pl.BlockSpec((BR, 1), lambda i: (i, 0)),
              pl.BlockSpec((1, D), lambda i: (0, 0)),
              pl.BlockSpec((D, D), lambda i: (0, 0))],
    out_specs=pl.BlockSpec((BR, D), lambda i: (i, 0)),
    out_shape=jax.ShapeDtypeStruct((NP, D), jnp.float32),
)


def _final_body(q_ref, dinv_ref, b2_ref, wp_ref, w3_ref, wp2_ref, wc_ref,
                b3_ref, bp_ref, bc_ref, out_ref, acc_ref):
    i = pl.program_id(0)
    d = dinv_ref[...]
    h2 = jnp.maximum((q_ref[0] + q_ref[1]) * d + b2_ref[...], 0.0)
    wsum = jnp.sum(wp_ref[...], axis=0)[:, None]
    u = d * (d + wsum)
    mask = (lax.broadcasted_iota(jnp.int32, (BR, 1), 0) + i * BR) < N
    u = jnp.where(mask, u, 0.0)
    contr = jnp.sum(u * h2, axis=0, keepdims=True)

    @pl.when(i == 0)
    def _():
        acc_ref[...] = contr

    @pl.when(i > 0)
    def _():
        acc_ref[...] += contr

    @pl.when(i == GRID - 1)
    def _():
        t = acc_ref[...]
        g0 = jnp.dot(t, w3_ref[...], precision=_HI,
                     preferred_element_type=jnp.float32) * (1.0 / N) + b3_ref[...]
        g1 = jnp.maximum(jnp.dot(g0, wp2_ref[...], precision=_HI,
                                 preferred_element_type=jnp.float32) + bp_ref[...], 0.0)
        out_ref[...] = jnp.dot(g1, wc_ref[...], precision=_HI,
                               preferred_element_type=jnp.float32) + bc_ref[...]


_final = pl.pallas_call(
    _final_body,
    grid=(GRID,),
    in_specs=[pl.BlockSpec((NC, BR, D), lambda i: (0, i, 0)),
              pl.BlockSpec((BR, 1), lambda i: (i, 0)),
              pl.BlockSpec((1, D), lambda i: (0, 0)),
              pl.BlockSpec((NW, BR), lambda i: (0, i)),
              pl.BlockSpec((D, D), lambda i: (0, 0)),
              pl.BlockSpec((D, D), lambda i: (0, 0)),
              pl.BlockSpec((D, 1), lambda i: (0, 0)),
              pl.BlockSpec((1, D), lambda i: (0, 0)),
              pl.BlockSpec((1, D), lambda i: (0, 0)),
              pl.BlockSpec((1, 1), lambda i: (0, 0))],
    out_specs=pl.BlockSpec((1, 1), lambda i: (0, 0)),
    out_shape=jax.ShapeDtypeStruct((1, 1), jnp.float32),
    scratch_shapes=[pltpu.VMEM((1, D), jnp.float32)],
)


def kernel(x, edge_index, W1, b1, W2, b2, W3, b3, Wp, bp, Wc, bc):
    src = edge_index[0]
    dst = edge_index[1]
    zerosd = jnp.zeros((NP, D), jnp.float32)
    padn = ((0, 0), (0, NP - N))

    x_pad = jnp.pad(x, ((0, NP - N), (0, 0)))
    degp = jnp.pad(_deg_kernel(dst)[:, 0, :], padn)

    z1, dinvc = _mm1(x_pad, degp, W1)
    dinv = dinvc[:N, 0]

    p, wp_ = _agg_w_kernel(z1, src, dst, dinv, zerosd)
    z2 = _mm2(p, dinvc, b1.reshape(1, D), W2)
    q = _agg_kernel(z2, src, dst, zerosd)

    wpad = jnp.pad(wp_[:, 0, :], padn)
    return _final(q, dinvc, b2.reshape(1, D), wpad, W3, Wp, Wc,
                  b3.reshape(1, D), bp.reshape(1, D), bc.reshape(1, 1))


# speedup vs baseline: 2.0004x; 2.0004x over previous
import functools

import jax
import jax.numpy as jnp
from jax import lax
from jax.experimental import pallas as pl
from jax.experimental.pallas import tpu as pltpu
from jax.experimental.pallas import tpu_sc as plsc

N = 10000
E = 320000
D = 128

NC, NS, L = 2, 16, 16
NW = NC * NS
EPW = E // NW
C = 80
NCHUNK = EPW // C
NPAIR = (NCHUNK - 1) // 2
NP = 10240
RPT = NP // NS

_mesh = plsc.VectorSubcoreMesh(core_axis_name="c", subcore_axis_name="s")
_params = pltpu.CompilerParams(needs_layout_passes=False)


@functools.partial(
    pl.kernel,
    out_type=jax.ShapeDtypeStruct((NW, 1, N), jnp.float32),
    mesh=_mesh,
    compiler_params=_params,
    scratch_types=[
        pltpu.VMEM((1, EPW), jnp.int32),
        pltpu.VMEM((1, N), jnp.float32),
    ],
)
def _deg_kernel(dst_hbm, out_hbm, idxv, acc):
    cid = lax.axis_index("c")
    sid = lax.axis_index("s")
    wid = sid * NC + cid
    pltpu.sync_copy(dst_hbm.at[wid], idxv)
    zf = jnp.zeros((L,), jnp.float32)

    def zinit(i, _):
        acc[0, pl.ds(i * L, L)] = zf
        return 0

    lax.fori_loop(0, N // L, zinit, 0)

    row0 = jnp.zeros((L,), jnp.int32)
    onesv = jnp.ones((L,), jnp.float32)

    def body(j, _):
        iv = idxv[0, pl.ds(j * L, L)]
        plsc.addupdate_scatter(acc, [row0, iv], onesv)
        return 0

    lax.fori_loop(0, EPW // L, body, 0)
    pltpu.sync_copy(acc, out_hbm.at[wid])


@functools.partial(
    pl.kernel,
    out_type=jax.ShapeDtypeStruct((NW, 1, N), jnp.float32),
    mesh=_mesh,
    compiler_params=_params,
    scratch_types=[
        pltpu.VMEM((1, EPW), jnp.int32),
        pltpu.VMEM((1, EPW), jnp.int32),
        pltpu.VMEM((N,), jnp.float32),
        pltpu.VMEM((1, N), jnp.float32),
    ],
)
def _w_kernel(src_hbm, dst_hbm, dinv_hbm, wout_hbm, srcf, dstf, dinv_v, wacc):
    cid = lax.axis_index("c")
    sid = lax.axis_index("s")
    wid = sid * NC + cid
    pltpu.sync_copy(src_hbm.at[wid], srcf)
    pltpu.sync_copy(dst_hbm.at[wid], dstf)
    pltpu.sync_copy(dinv_hbm, dinv_v)
    zf = jnp.zeros((L,), jnp.float32)

    def zinit(i, _):
        wacc[0, pl.ds(i * L, L)] = zf
        return 0

    lax.fori_loop(0, N // L, zinit, 0)

    row0 = jnp.zeros((L,), jnp.int32)

    def body(j, _):
        iv_d = dstf[0, pl.ds(j * L, L)]
        iv_s = srcf[0, pl.ds(j * L, L)]
        dv = plsc.load_gather(dinv_v, [iv_d])
        plsc.addupdate_scatter(wacc, [row0, iv_s], dv)
        return 0

    lax.fori_loop(0, EPW // L, body, 0)
    pltpu.sync_copy(wacc, wout_hbm.at[wid])


@functools.partial(
    pl.kernel,
    out_type=jax.ShapeDtypeStruct((NC, NP, D), jnp.float32),
    mesh=_mesh,
    compiler_params=_params,
    scratch_types=[
        pltpu.VMEM((C,), jnp.int32),
        pltpu.VMEM((C,), jnp.int32),
        pltpu.VMEM((C,), jnp.int32),
        pltpu.VMEM((C,), jnp.int32),
        pltpu.VMEM((C, D), jnp.float32),
        pltpu.VMEM((C, D), jnp.float32),
        pltpu.VMEM_SHARED((NP, D), jnp.float32),
        pltpu.SemaphoreType.DMA,
        pltpu.SemaphoreType.DMA,
        pltpu.SemaphoreType.DMA,
        pltpu.SemaphoreType.DMA,
    ],
)
def _agg_kernel(z_hbm, src_hbm, dst_hbm, zeros_hbm, out_hbm,
                is0, is1, id0, id1, rows0, rows1, acc,
                semi0, semi1, semr0, semr1):
    cid = lax.axis_index("c")
    sid = lax.axis_index("s")
    wid = sid * NC + cid
    sl = pl.ds(sid * RPT, RPT)
    @pl.when(cid == 0)
    def _():
        pltpu.sync_copy(z_hbm.at[sl], acc.at[sl])

    @pl.when(cid != 0)
    def _():
        pltpu.sync_copy(zeros_hbm.at[sl], acc.at[sl])

    isb = (is0, is1)
    idb = (id0, id1)
    rows = (rows0, rows1)
    semi = (semi0, semi1)
    semr = (semr0, semr1)
    e0 = wid * EPW

    plsc.subcore_barrier()

    pltpu.sync_copy(src_hbm.at[pl.ds(e0, C)], is0)
    pltpu.sync_copy(dst_hbm.at[pl.ds(e0, C)], id0)
    pltpu.async_copy(src_hbm.at[pl.ds(e0 + C, C)], is1, semi1)
    pltpu.async_copy(dst_hbm.at[pl.ds(e0 + C, C)], id1, semi1)
    pltpu.async_copy(z_hbm.at[is0], rows0, semr0)

    def step(g, b):
        nb = 1 - b
        pltpu.make_async_copy(src_hbm.at[pl.ds(e0, C)], isb[nb], semi[nb]).wait()
        pltpu.make_async_copy(dst_hbm.at[pl.ds(e0, C)], idb[nb], semi[nb]).wait()
        pltpu.async_copy(z_hbm.at[isb[nb]], rows[nb], semr[nb])
        pltpu.make_async_copy(z_hbm.at[isb[b]], rows[b], semr[b]).wait()
        pltpu.sync_copy(rows[b], acc.at[idb[b]], add=True)

        @pl.when(g + 2 < NCHUNK)
        def _():
            base = e0 + (g + 2) * C
            pltpu.async_copy(src_hbm.at[pl.ds(base, C)], isb[b], semi[b])
            pltpu.async_copy(dst_hbm.at[pl.ds(base, C)], idb[b], semi[b])

    def body(i, _):
        step(2 * i, 0)
        step(2 * i + 1, 1)
        return 0

    lax.fori_loop(0, NPAIR, body, 0)
    for g in range(2 * NPAIR, NCHUNK):
        b = g % 2
        pltpu.make_async_copy(z_hbm.at[isb[b]], rows[b], semr[b]).wait()
        pltpu.sync_copy(rows[b], acc.at[idb[b]], add=True)

    plsc.subcore_barrier()
    pltpu.sync_copy(acc.at[sl], out_hbm.at[cid, sl])


BR = 640
GRID = NP // BR
_HI = lax.Precision.HIGHEST


def _mm1_body(x_ref, degp_ref, w_ref, z_ref, dinv_ref):
    d = lax.rsqrt(jnp.sum(degp_ref[...], axis=0) + 1.0)[:, None]
    z_ref[...] = jnp.dot(x_ref[...], w_ref[...], precision=_HI,
                         preferred_element_type=jnp.float32) * d
    dinv_ref[...] = d


_mm1 = pl.pallas_call(
    _mm1_body,
    grid=(GRID,),
    in_specs=[pl.BlockSpec((BR, D), lambda i: (i, 0)),
              pl.BlockSpec((NW, BR), lambda i: (0, i)),
              pl.BlockSpec((D, D), lambda i: (0, 0))],
    out_specs=[pl.BlockSpec((BR, D), lambda i: (i, 0)),
               pl.BlockSpec((BR, 1), lambda i: (i, 0))],
    out_shape=[jax.ShapeDtypeStruct((NP, D), jnp.float32),
               jax.ShapeDtypeStruct((NP, 1), jnp.float32)],
)


def _mm2_body(p_ref, dinv_ref, b1_ref, w_ref, z_ref):
    d = dinv_ref[...]
    h1 = jnp.maximum((p_ref[0] + p_ref[1]) * d + b1_ref[...], 0.0)
    z_ref[...] = jnp.dot(h1, w_ref[...], precision=_HI,
                         preferred_element_type=jnp.float32) * d


_mm2 = pl.pallas_call(
    _mm2_body,
    grid=(GRID,),
    in_specs=[pl.BlockSpec((NC, BR, D), lambda i: (0, i, 0)),
              pl.BlockSpec((BR, 1), lambda i: (i, 0)),
              pl.BlockSpec((1, D), lambda i: (0, 0)),
              pl.BlockSpec((D, D), lambda i: (0, 0))],
    out_specs=pl.BlockSpec((BR, D), lambda i: (i, 0)),
    out_shape=jax.ShapeDtypeStruct((NP, D), jnp.float32),
)


def _final_body(q_ref, dinv_ref, b2_ref, wp_ref, w3_ref, wp2_ref, wc_ref,
                b3_ref, bp_ref, bc_ref, out_ref, acc_ref):
    i = pl.program_id(0)
    d = dinv_ref[...]
    h2 = jnp.maximum((q_ref[0] + q_ref[1]) * d + b2_ref[...], 0.0)
    wsum = jnp.sum(wp_ref[...], axis=0)[:, None]
    u = d * (d + wsum)
    mask = (lax.broadcasted_iota(jnp.int32, (BR, 1), 0) + i * BR) < N
    u = jnp.where(mask, u, 0.0)
    contr = jnp.sum(u * h2, axis=0, keepdims=True)

    @pl.when(i == 0)
    def _():
        acc_ref[...] = contr

    @pl.when(i > 0)
    def _():
        acc_ref[...] += contr

    @pl.when(i == GRID - 1)
    def _():
        t = acc_ref[...]
        g0 = jnp.dot(t, w3_ref[...], precision=_HI,
                     preferred_element_type=jnp.float32) * (1.0 / N) + b3_ref[...]
        g1 = jnp.maximum(jnp.dot(g0, wp2_ref[...], precision=_HI,
                                 preferred_element_type=jnp.float32) + bp_ref[...], 0.0)
        out_ref[...] = jnp.dot(g1, wc_ref[...], precision=_HI,
                               preferred_element_type=jnp.float32) + bc_ref[...]


_final = pl.pallas_call(
    _final_body,
    grid=(GRID,),
    in_specs=[pl.BlockSpec((NC, BR, D), lambda i: (0, i, 0)),
              pl.BlockSpec((BR, 1), lambda i: (i, 0)),
              pl.BlockSpec((1, D), lambda i: (0, 0)),
              pl.BlockSpec((NW, BR), lambda i: (0, i)),
              pl.BlockSpec((D, D), lambda i: (0, 0)),
              pl.BlockSpec((D, D), lambda i: (0, 0)),
              pl.BlockSpec((D, 1), lambda i: (0, 0)),
              pl.BlockSpec((1, D), lambda i: (0, 0)),
              pl.BlockSpec((1, D), lambda i: (0, 0)),
              pl.BlockSpec((1, 1), lambda i: (0, 0))],
    out_specs=pl.BlockSpec((1, 1), lambda i: (0, 0)),
    out_shape=jax.ShapeDtypeStruct((1, 1), jnp.float32),
    scratch_shapes=[pltpu.VMEM((1, D), jnp.float32)],
)


def kernel(x, edge_index, W1, b1, W2, b2, W3, b3, Wp, bp, Wc, bc):
    src = edge_index[0]
    dst = edge_index[1]
    src1 = src.reshape(NW, 1, EPW)
    dst1 = dst.reshape(NW, 1, EPW)
    zerosd = jnp.zeros((NP, D), jnp.float32)
    padn = ((0, 0), (0, NP - N))

    x_pad = jnp.pad(x, ((0, NP - N), (0, 0)))
    degp = jnp.pad(_deg_kernel(dst1)[:, 0, :], padn)

    z1, dinvc = _mm1(x_pad, degp, W1)
    dinv = dinvc[:N, 0]

    wp_ = _w_kernel(src1, dst1, dinv)
    p = _agg_kernel(z1, src, dst, zerosd)
    z2 = _mm2(p, dinvc, b1.reshape(1, D), W2)
    q = _agg_kernel(z2, src, dst, zerosd)

    wpad = jnp.pad(wp_[:, 0, :], padn)
    return _final(q, dinvc, b2.reshape(1, D), wpad, W3, Wp, Wc,
                  b3.reshape(1, D), bp.reshape(1, D), bc.reshape(1, 1))
